# Initial kernel scaffold; baseline (speedup 1.0000x reference)
import jax
import jax.numpy as jnp
from jax.experimental import pallas as pl


def kernel(x, w1, b1, g1, be1, alpha, w2, b2, g2, be2):
    raise NotImplementedError("write your pallas kernel here")


# speedup vs baseline: 1.1031x; 1.1031x over previous
import functools

import jax
import jax.numpy as jnp
from jax.experimental import pallas as pl
from jax.experimental.pallas import tpu as pltpu

_BF = jnp.bfloat16


def _lane_roll(a, d):
    L = a.shape[-1]
    d = d % L
    if d == 0:
        return a
    return jnp.concatenate([a[..., d:], a[..., :d]], axis=-1)


def _taps(ab, H, W):
    HW = H * W
    pos = jax.lax.broadcasted_iota(jnp.int32, (1, HW), 1)
    hpos = pos // W
    wpos = pos - hpos * W
    parts = []
    for dy in (-1, 0, 1):
        for dx in (-1, 0, 1):
            delta = dy * W + dx
            s = _lane_roll(ab, delta)
            conds = []
            if dy == -1:
                conds.append(hpos >= 1)
            elif dy == 1:
                conds.append(hpos <= H - 2)
            if dx == -1:
                conds.append(wpos >= 1)
            elif dx == 1:
                conds.append(wpos <= W - 2)
            if conds:
                valid = functools.reduce(lambda u, v: u & v, conds)
                s = jnp.where(valid, s, jnp.zeros((), _BF))
            parts.append(s)
    return jnp.concatenate(parts, axis=0)


def _conv9(ab, w_ref, H, W):
    big = _taps(ab, H, W)
    return jnp.dot(w_ref[...], big, preferred_element_type=jnp.float32)


def _stage1_kernel(H, W, C, B, x_ref, w_ref, y_ref, stats_ref):
    s = jnp.zeros((C, 1), jnp.float32)
    q = jnp.zeros((C, 1), jnp.float32)
    for b in range(B):
        ab = x_ref[b].astype(_BF)
        acc = _conv9(ab, w_ref, H, W)
        y_ref[b, :, :] = acc.astype(_BF)
        s = s + jnp.sum(acc, axis=1, keepdims=True)
        q = q + jnp.sum(acc * acc, axis=1, keepdims=True)
    stats_ref[:, 0:1] = s
    stats_ref[:, 1:2] = q


def _stage2_kernel(H, W, C, B, y1_ref, w_ref, scale_ref, shift_ref,
                   alpha_ref, y_ref, stats_ref):
    s = jnp.zeros((C, 1), jnp.float32)
    q = jnp.zeros((C, 1), jnp.float32)
    for b in range(B):
        z = y1_ref[b] * scale_ref[...] + shift_ref[...]
        ab = jnp.where(z >= 0, z, alpha_ref[...] * z)
        acc = _conv9(ab, w_ref, H, W)
        y_ref[b, :, :] = acc.astype(_BF)
        s = s + jnp.sum(acc, axis=1, keepdims=True)
        q = q + jnp.sum(acc * acc, axis=1, keepdims=True)
    stats_ref[:, 0:1] = s
    stats_ref[:, 1:2] = q


def _final_kernel(x_ref, y_ref, scale_ref, shift_ref, out_ref):
    out_ref[...] = (x_ref[...]
                    + y_ref[...].astype(jnp.float32) * scale_ref[...]
                    + shift_ref[...])


def _fold_bn(stats, gamma, beta, count, eps):
    s = jnp.sum(stats[:, :, 0], axis=0)
    q = jnp.sum(stats[:, :, 1], axis=0)
    mean = s / count
    var = jnp.maximum(q / count - mean * mean, 0.0)
    scale = gamma.astype(jnp.float32) * jax.lax.rsqrt(var + eps)
    shift = beta.astype(jnp.float32) - mean * scale
    return scale, shift


def _prep_w(w_oihw, C):
    return jnp.transpose(w_oihw, (0, 2, 3, 1)).reshape(C, 9 * C).astype(_BF)


def kernel(x, w1, b1, g1, be1, alpha, w2, b2, g2, be2, eps=1e-5):
    x = x.astype(jnp.float32)
    N, C, H, W = x.shape
    HW = H * W
    B = 2
    BF_ = 4
    G = N // B
    GF = N // BF_
    count = N * HW

    xg = x.reshape(G, B, C, HW)
    w1c = _prep_w(w1, C)
    w2c = _prep_w(w2, C)
    alpha_c = jnp.broadcast_to(
        alpha.astype(_BF).reshape(-1), (C,)).reshape(C, 1)

    img = lambda i: (i, 0, 0, 0)
    img3 = lambda i: (i, 0, 0)
    rep2 = lambda i: (0, 0)
    cparams = pltpu.CompilerParams(
        dimension_semantics=("parallel",),
        vmem_limit_bytes=48 << 20,
    )

    stage1 = pl.pallas_call(
        functools.partial(_stage1_kernel, H, W, C, B),
        grid=(G,),
        in_specs=[
            pl.BlockSpec((None, B, C, HW), img),
            pl.BlockSpec((C, 9 * C), rep2),
        ],
        out_specs=(
            pl.BlockSpec((None, B, C, HW), img),
            pl.BlockSpec((None, C, 2), img3),
        ),
        out_shape=(
            jax.ShapeDtypeStruct((G, B, C, HW), _BF),
            jax.ShapeDtypeStruct((G, C, 2), jnp.float32),
        ),
        compiler_params=cparams,
    )

    stage2 = pl.pallas_call(
        functools.partial(_stage2_kernel, H, W, C, B),
        grid=(G,),
        in_specs=[
            pl.BlockSpec((None, B, C, HW), img),
            pl.BlockSpec((C, 9 * C), rep2),
            pl.BlockSpec((C, 1), rep2),
            pl.BlockSpec((C, 1), rep2),
            pl.BlockSpec((C, 1), rep2),
        ],
        out_specs=(
            pl.BlockSpec((None, B, C, HW), img),
            pl.BlockSpec((None, C, 2), img3),
        ),
        out_shape=(
            jax.ShapeDtypeStruct((G, B, C, HW), _BF),
            jax.ShapeDtypeStruct((G, C, 2), jnp.float32),
        ),
        compiler_params=cparams,
    )

    finalize = pl.pallas_call(
        _final_kernel,
        grid=(GF,),
        in_specs=[
            pl.BlockSpec((None, BF_, C, HW), img),
            pl.BlockSpec((None, BF_, C, HW), img),
            pl.BlockSpec((1, C, 1), lambda i: (0, 0, 0)),
            pl.BlockSpec((1, C, 1), lambda i: (0, 0, 0)),
        ],
        out_specs=pl.BlockSpec((None, BF_, C, HW), img),
        out_shape=jax.ShapeDtypeStruct((GF, BF_, C, HW), jnp.float32),
        compiler_params=cparams,
    )

    y1, st1 = stage1(xg, w1c)
    scale1, shift1 = _fold_bn(st1, g1, be1, count, eps)

    y2, st2 = stage2(y1, w2c,
                     scale1.astype(_BF).reshape(C, 1),
                     shift1.astype(_BF).reshape(C, 1),
                     alpha_c)
    scale2, shift2 = _fold_bn(st2, g2, be2, count, eps)

    out = finalize(xg.reshape(GF, BF_, C, HW),
                   y2.reshape(GF, BF_, C, HW),
                   scale2.reshape(1, C, 1),
                   shift2.reshape(1, C, 1))
    return out.reshape(N, C, H, W)


# speedup vs baseline: 1.2394x; 1.1236x over previous
import functools

import jax
import jax.numpy as jnp
from jax.experimental import pallas as pl
from jax.experimental.pallas import tpu as pltpu

_BF = jnp.bfloat16


def _lane_roll(a, d):
    L = a.shape[-1]
    d = d % L
    if d == 0:
        return a
    return jnp.concatenate([a[..., d:], a[..., :d]], axis=-1)


def _taps(ab, H, W):
    HW = H * W
    pos = jax.lax.broadcasted_iota(jnp.int32, (1, HW), 1)
    hpos = pos // W
    wpos = pos - hpos * W
    parts = []
    for dy in (-1, 0, 1):
        for dx in (-1, 0, 1):
            delta = dy * W + dx
            s = _lane_roll(ab, delta)
            conds = []
            if dy == -1:
                conds.append(hpos >= 1)
            elif dy == 1:
                conds.append(hpos <= H - 2)
            if dx == -1:
                conds.append(wpos >= 1)
            elif dx == 1:
                conds.append(wpos <= W - 2)
            if conds:
                valid = functools.reduce(lambda u, v: u & v, conds)
                s = jnp.where(valid, s, jnp.zeros((), _BF))
            parts.append(s)
    return jnp.concatenate(parts, axis=0)


def _conv9(ab, w_ref, H, W):
    big = _taps(ab, H, W)
    return jnp.dot(w_ref[...], big, preferred_element_type=jnp.float32)


def _fold(s, q, gamma, beta, count, eps):
    mean = s / count
    var = jnp.maximum(q / count - mean * mean, 0.0)
    scale = gamma * jax.lax.rsqrt(var + eps)
    shift = beta - mean * scale
    return scale, shift


def _mono_kernel(H, W, C, B, G, count, eps,
                 x_ref, w1_ref, w2_ref, pm_ref, out_ref,
                 y1_scr, y2_scr, st1_scr, st2_scr, sc1_scr, sc2_scr):
    i = pl.program_id(0)
    HW = H * W

    @pl.when(i == 0)
    def _init():
        st1_scr[...] = jnp.zeros_like(st1_scr)
        st2_scr[...] = jnp.zeros_like(st2_scr)

    @pl.when(i < G)
    def _phase_a():
        s = jnp.zeros((C, 1), jnp.float32)
        q = jnp.zeros((C, 1), jnp.float32)
        for b in range(B):
            ab = x_ref[b].astype(_BF)
            acc = _conv9(ab, w1_ref, H, W)
            y1_scr[i, b] = acc.astype(_BF)
            s = s + jnp.sum(acc, axis=1, keepdims=True)
            q = q + jnp.sum(acc * acc, axis=1, keepdims=True)
        st1_scr[:, 0:1] += s
        st1_scr[:, 1:2] += q

    @pl.when(i == G)
    def _fold1():
        pmt = jnp.transpose(pm_ref[...])
        scale, shift = _fold(st1_scr[:, 0:1], st1_scr[:, 1:2],
                             pmt[:, 0:1], pmt[:, 1:2], count, eps)
        sc1_scr[:, 0:1] = scale.astype(_BF)
        sc1_scr[:, 1:2] = shift.astype(_BF)
        sc1_scr[:, 2:3] = pmt[:, 4:5].astype(_BF)

    @pl.when((i >= G) & (i < 2 * G))
    def _phase_b():
        j = i - G
        scale = sc1_scr[:, 0:1]
        shift = sc1_scr[:, 1:2]
        al = sc1_scr[:, 2:3]
        s = jnp.zeros((C, 1), jnp.float32)
        q = jnp.zeros((C, 1), jnp.float32)
        for b in range(B):
            z = y1_scr[j, b] * scale + shift
            ab = jnp.where(z >= 0, z, al * z)
            acc = _conv9(ab, w2_ref, H, W)
            y2_scr[j, b] = acc.astype(_BF)
            s = s + jnp.sum(acc, axis=1, keepdims=True)
            q = q + jnp.sum(acc * acc, axis=1, keepdims=True)
        st2_scr[:, 0:1] += s
        st2_scr[:, 1:2] += q

    @pl.when(i == 2 * G)
    def _fold2():
        pmt = jnp.transpose(pm_ref[...])
        scale, shift = _fold(st2_scr[:, 0:1], st2_scr[:, 1:2],
                             pmt[:, 2:3], pmt[:, 3:4], count, eps)
        sc2_scr[:, 0:1] = scale
        sc2_scr[:, 1:2] = shift

    @pl.when(i >= 2 * G)
    def _phase_c():
        k = i - 2 * G
        scale = sc2_scr[:, 0:1].reshape(1, C, 1)
        shift = sc2_scr[:, 1:2].reshape(1, C, 1)
        out_ref[...] = (x_ref[...]
                        + y2_scr[k].astype(jnp.float32) * scale
                        + shift)


def _prep_w(w_oihw, C):
    return jnp.transpose(w_oihw, (0, 2, 3, 1)).reshape(C, 9 * C).astype(_BF)


def kernel(x, w1, b1, g1, be1, alpha, w2, b2, g2, be2, eps=1e-5):
    x = x.astype(jnp.float32)
    N, C, H, W = x.shape
    HW = H * W
    B = 2
    G = N // B
    count = float(N * HW)

    xg = x.reshape(G, B, C, HW)
    w1c = _prep_w(w1, C)
    w2c = _prep_w(w2, C)
    zc = jnp.zeros_like(g1)
    pm = jnp.stack([g1, be1, g2, be2,
                    jnp.broadcast_to(alpha, g1.shape), zc, zc, zc]
                   ).astype(jnp.float32)

    x_map = lambda i: (jnp.where(i < G, i, jnp.where(i < 2 * G, G - 1,
                                                     i - 2 * G)), 0, 0, 0)
    out_map = lambda i: (jnp.where(i < 2 * G, 0, i - 2 * G), 0, 0, 0)
    rep2 = lambda i: (0, 0)

    mono = pl.pallas_call(
        functools.partial(_mono_kernel, H, W, C, B, G, count, eps),
        grid=(3 * G,),
        in_specs=[
            pl.BlockSpec((None, B, C, HW), x_map),
            pl.BlockSpec((C, 9 * C), rep2),
            pl.BlockSpec((C, 9 * C), rep2),
            pl.BlockSpec((8, C), rep2),
        ],
        out_specs=pl.BlockSpec((None, B, C, HW), out_map),
        out_shape=jax.ShapeDtypeStruct((G, B, C, HW), jnp.float32),
        scratch_shapes=[
            pltpu.VMEM((G, B, C, HW), _BF),
            pltpu.VMEM((G, B, C, HW), _BF),
            pltpu.VMEM((C, 2), jnp.float32),
            pltpu.VMEM((C, 2), jnp.float32),
            pltpu.VMEM((C, 4), _BF),
            pltpu.VMEM((C, 2), jnp.float32),
        ],
        compiler_params=pltpu.CompilerParams(
            dimension_semantics=("arbitrary",),
            vmem_limit_bytes=56 << 20,
        ),
    )

    out = mono(xg, w1c, w2c, pm)
    return out.reshape(N, C, H, W)


# speedup vs baseline: 1.3739x; 1.1085x over previous
import functools

import jax
import jax.numpy as jnp
from jax.experimental import pallas as pl
from jax.experimental.pallas import tpu as pltpu

_BF = jnp.bfloat16


def _shift_zfill(a, d):
    L = a.shape[-1]
    z = jnp.zeros(a.shape[:-1] + (abs(d),), a.dtype)
    if d > 0:
        return jnp.concatenate([a[..., d:], z], axis=-1)
    return jnp.concatenate([z, a[..., :L + d]], axis=-1)


def _taps(ab, H, W):
    HW = H * W
    pos = jax.lax.broadcasted_iota(jnp.int32, (1, HW), 1)
    wpos = pos % W
    parts = []
    for dy in (-1, 0, 1):
        for dx in (-1, 0, 1):
            delta = dy * W + dx
            s = ab if delta == 0 else _shift_zfill(ab, delta)
            if dx == -1:
                s = jnp.where(wpos >= 1, s, jnp.zeros((), _BF))
            elif dx == 1:
                s = jnp.where(wpos <= W - 2, s, jnp.zeros((), _BF))
            parts.append(s)
    return parts


def _conv9(ab, w_ref, r0, C, H, W):
    parts = _taps(ab, H, W)
    acc = None
    for t0, t1 in ((0, 2), (2, 4), (4, 6), (6, 8), (8, 9)):
        seg = parts[t0] if t1 == t0 + 1 else jnp.concatenate(
            parts[t0:t1], axis=0)
        wseg = w_ref[r0:r0 + C, t0 * C:t1 * C]
        d = jnp.dot(wseg, seg, preferred_element_type=jnp.float32)
        acc = d if acc is None else acc + d
    return acc


def _fold(s, q, gamma, beta, count, eps):
    mean = s / count
    var = jnp.maximum(q / count - mean * mean, 0.0)
    scale = gamma * jax.lax.rsqrt(var + eps)
    shift = beta - mean * scale
    return scale, shift


def _mono_kernel(H, W, C, B, G, count, eps,
                 x_ref, w_ref, out_ref,
                 y1_scr, y2_scr, xb_scr, st1_scr, st2_scr, sc1_scr, sc2_scr):
    i = pl.program_id(0)

    @pl.when(i == 0)
    def _init():
        st1_scr[...] = jnp.zeros_like(st1_scr)
        st2_scr[...] = jnp.zeros_like(st2_scr)

    @pl.when(i < G)
    def _phase_a():
        s = jnp.zeros((C, 1), jnp.float32)
        q = jnp.zeros((C, 1), jnp.float32)
        for b in range(B):
            ab = x_ref[b].astype(_BF)
            xb_scr[i, b] = ab
            acc = _conv9(ab, w_ref, 0, C, H, W)
            y1_scr[i, b] = acc.astype(_BF)
            s = s + jnp.sum(acc, axis=1, keepdims=True)
            q = q + jnp.sum(acc * acc, axis=1, keepdims=True)
        st1_scr[:, 0:1] += s
        st1_scr[:, 1:2] += q

    @pl.when(i == G)
    def _fold1():
        pmt = jnp.transpose(
            w_ref[2 * C:2 * C + 8, 0:C].astype(jnp.float32))
        scale, shift = _fold(st1_scr[:, 0:1], st1_scr[:, 1:2],
                             pmt[:, 0:1], pmt[:, 1:2], count, eps)
        sc1_scr[:, 0:1] = scale.astype(_BF)
        sc1_scr[:, 1:2] = shift.astype(_BF)
        sc1_scr[:, 2:3] = pmt[:, 4:5].astype(_BF)

    @pl.when((i >= G) & (i < 2 * G))
    def _phase_b():
        j = i - G
        scale = sc1_scr[:, 0:1]
        shift = sc1_scr[:, 1:2]
        al = sc1_scr[:, 2:3]
        s = jnp.zeros((C, 1), jnp.float32)
        q = jnp.zeros((C, 1), jnp.float32)
        for b in range(B):
            z = y1_scr[j, b] * scale + shift
            ab = jnp.where(z >= 0, z, al * z)
            acc = _conv9(ab, w_ref, C, C, H, W)
            y2_scr[j, b] = acc.astype(_BF)
            s = s + jnp.sum(acc, axis=1, keepdims=True)
            q = q + jnp.sum(acc * acc, axis=1, keepdims=True)
        st2_scr[:, 0:1] += s
        st2_scr[:, 1:2] += q

    @pl.when(i == 2 * G)
    def _fold2():
        pmt = jnp.transpose(
            w_ref[2 * C:2 * C + 8, 0:C].astype(jnp.float32))
        scale, shift = _fold(st2_scr[:, 0:1], st2_scr[:, 1:2],
                             pmt[:, 2:3], pmt[:, 3:4], count, eps)
        sc2_scr[:, 0:1] = scale
        sc2_scr[:, 1:2] = shift

    @pl.when(i >= 2 * G)
    def _phase_c():
        k = i - 2 * G
        scale = sc2_scr[:, 0:1].reshape(1, C, 1)
        shift = sc2_scr[:, 1:2].reshape(1, C, 1)
        out_ref[...] = (xb_scr[k].astype(jnp.float32)
                        + y2_scr[k].astype(jnp.float32) * scale
                        + shift)


def _prep_w(w_oihw, C):
    return jnp.transpose(w_oihw, (0, 2, 3, 1)).reshape(C, 9 * C)


def kernel(x, w1, b1, g1, be1, alpha, w2, b2, g2, be2, eps=1e-5):
    x = x.astype(jnp.float32)
    N, C, H, W = x.shape
    HW = H * W
    B = 2
    G = N // B
    count = float(N * HW)

    xg = x.reshape(G, B, C, HW)
    pm = jnp.stack([g1, be1, g2, be2,
                    jnp.broadcast_to(alpha, g1.shape),
                    jnp.zeros_like(g1), jnp.zeros_like(g1),
                    jnp.zeros_like(g1)]).astype(jnp.float32)
    w_all = jnp.concatenate([
        _prep_w(w1, C),
        _prep_w(w2, C),
        jnp.pad(pm, ((0, 0), (0, 8 * C))),
    ], axis=0).astype(_BF)

    x_map = lambda i: (jnp.where(i < G, i, G - 1), 0, 0, 0)
    out_map = lambda i: (jnp.where(i < 2 * G, 0, i - 2 * G), 0, 0, 0)

    mono = pl.pallas_call(
        functools.partial(_mono_kernel, H, W, C, B, G, count, eps),
        grid=(3 * G,),
        in_specs=[
            pl.BlockSpec((None, B, C, HW), x_map),
            pl.BlockSpec((2 * C + 8, 9 * C), lambda i: (0, 0)),
        ],
        out_specs=pl.BlockSpec((None, B, C, HW), out_map),
        out_shape=jax.ShapeDtypeStruct((G, B, C, HW), jnp.float32),
        scratch_shapes=[
            pltpu.VMEM((G, B, C, HW), _BF),
            pltpu.VMEM((G, B, C, HW), _BF),
            pltpu.VMEM((G, B, C, HW), _BF),
            pltpu.VMEM((C, 2), jnp.float32),
            pltpu.VMEM((C, 2), jnp.float32),
            pltpu.VMEM((C, 4), _BF),
            pltpu.VMEM((C, 2), jnp.float32),
        ],
        compiler_params=pltpu.CompilerParams(
            dimension_semantics=("arbitrary",),
            vmem_limit_bytes=58 << 20,
        ),
    )

    out = mono(xg, w_all)
    return out.reshape(N, C, H, W)


# speedup vs baseline: 1.4657x; 1.0668x over previous
import functools

import jax
import jax.numpy as jnp
from jax.experimental import pallas as pl
from jax.experimental.pallas import tpu as pltpu

_BF = jnp.bfloat16


def _shift_zfill(a, d):
    L = a.shape[-1]
    z = jnp.zeros(a.shape[:-1] + (abs(d),), a.dtype)
    if d > 0:
        return jnp.concatenate([a[..., d:], z], axis=-1)
    return jnp.concatenate([z, a[..., :L + d]], axis=-1)


def _taps(ab, H, W):
    HW = H * W
    pos = jax.lax.broadcasted_iota(jnp.int32, (1, HW), 1)
    wpos = pos % W
    parts = []
    for dy in (-1, 0, 1):
        for dx in (-1, 0, 1):
            delta = dy * W + dx
            s = ab if delta == 0 else _shift_zfill(ab, delta)
            if dx == -1:
                s = jnp.where(wpos >= 1, s, jnp.zeros((), _BF))
            elif dx == 1:
                s = jnp.where(wpos <= W - 2, s, jnp.zeros((), _BF))
            parts.append(s)
    return parts


def _conv9(ab, w_ref, r0, C, H, W):
    parts = _taps(ab, H, W)
    acc = None
    for t0, t1 in ((0, 2), (2, 4), (4, 6), (6, 8), (8, 9)):
        seg = parts[t0] if t1 == t0 + 1 else jnp.concatenate(
            parts[t0:t1], axis=0)
        wseg = w_ref[r0:r0 + C, t0 * C:t1 * C]
        d = jnp.dot(wseg, seg, preferred_element_type=jnp.float32)
        acc = d if acc is None else acc + d
    return acc


def _fold(s, q, gamma, beta, count, eps):
    mean = s / count
    var = jnp.maximum(q / count - mean * mean, 0.0)
    scale = gamma * jax.lax.rsqrt(var + eps)
    shift = beta - mean * scale
    return scale, shift


def _mono_kernel(H, W, C, B, G, count, eps,
                 x_ref, w_ref, out_ref,
                 y1_scr, y2_scr, xb_scr, st1_scr, st2_scr, sc1_scr, sc2_scr):
    i = pl.program_id(0)

    @pl.when(i == 0)
    def _init():
        st1_scr[...] = jnp.zeros_like(st1_scr)
        st2_scr[...] = jnp.zeros_like(st2_scr)

    @pl.when(i < G)
    def _phase_a():
        s = jnp.zeros((C, 1), jnp.float32)
        q = jnp.zeros((C, 1), jnp.float32)
        for b in range(B):
            ab = x_ref[b].astype(_BF)
            xb_scr[i, b] = ab
            acc = _conv9(ab, w_ref, 0, C, H, W)
            y1_scr[i, b] = acc.astype(_BF)
            s = s + jnp.sum(acc, axis=1, keepdims=True)
            q = q + jnp.sum(acc * acc, axis=1, keepdims=True)
        st1_scr[:, 0:1] += s
        st1_scr[:, 1:2] += q

    @pl.when(i == G)
    def _fold1():
        pmt = jnp.transpose(
            w_ref[2 * C:2 * C + 8, 0:C].astype(jnp.float32))
        scale, shift = _fold(st1_scr[:, 0:1], st1_scr[:, 1:2],
                             pmt[:, 0:1], pmt[:, 1:2], count, eps)
        sc1_scr[:, 0:1] = scale.astype(_BF)
        sc1_scr[:, 1:2] = shift.astype(_BF)
        sc1_scr[:, 2:3] = pmt[:, 4:5].astype(_BF)

    @pl.when((i >= G) & (i < 2 * G))
    def _phase_b():
        j = i - G
        scale = sc1_scr[:, 0:1]
        shift = sc1_scr[:, 1:2]
        al = sc1_scr[:, 2:3]
        s = jnp.zeros((C, 1), jnp.float32)
        q = jnp.zeros((C, 1), jnp.float32)
        for b in range(B):
            z = y1_scr[j, b] * scale + shift
            ab = jnp.where(z >= 0, z, al * z)
            acc = _conv9(ab, w_ref, C, C, H, W)
            y2_scr[j, b] = acc.astype(_BF)
            s = s + jnp.sum(acc, axis=1, keepdims=True)
            q = q + jnp.sum(acc * acc, axis=1, keepdims=True)
        st2_scr[:, 0:1] += s
        st2_scr[:, 1:2] += q

    @pl.when(i == 2 * G)
    def _fold2():
        pmt = jnp.transpose(
            w_ref[2 * C:2 * C + 8, 0:C].astype(jnp.float32))
        scale, shift = _fold(st2_scr[:, 0:1], st2_scr[:, 1:2],
                             pmt[:, 2:3], pmt[:, 3:4], count, eps)
        sc2_scr[:, 0:1] = scale
        sc2_scr[:, 1:2] = shift

    @pl.when(i >= 2 * G)
    def _phase_c():
        k = i - 2 * G
        scale = sc2_scr[:, 0:1].reshape(1, C, 1)
        shift = sc2_scr[:, 1:2].reshape(1, C, 1)
        out_ref[...] = (xb_scr[k].astype(jnp.float32)
                        + y2_scr[k].astype(jnp.float32) * scale
                        + shift)


def _prep_w(w_oihw, C):
    return jnp.transpose(w_oihw, (0, 2, 3, 1)).reshape(C, 9 * C)


def kernel(x, w1, b1, g1, be1, alpha, w2, b2, g2, be2, eps=1e-5):
    x = x.astype(jnp.float32)
    N, C, H, W = x.shape
    HW = H * W
    B = 4
    G = N // B
    count = float(N * HW)

    xg = x.reshape(G, B, C, HW)
    pm = jnp.stack([g1, be1, g2, be2,
                    jnp.broadcast_to(alpha, g1.shape),
                    jnp.zeros_like(g1), jnp.zeros_like(g1),
                    jnp.zeros_like(g1)]).astype(jnp.float32)
    w_all = jnp.concatenate([
        _prep_w(w1, C),
        _prep_w(w2, C),
        jnp.pad(pm, ((0, 0), (0, 8 * C))),
    ], axis=0).astype(_BF)

    x_map = lambda i: (jnp.where(i < G, i, G - 1), 0, 0, 0)
    out_map = lambda i: (jnp.where(i < 2 * G, 0, i - 2 * G), 0, 0, 0)

    mono = pl.pallas_call(
        functools.partial(_mono_kernel, H, W, C, B, G, count, eps),
        grid=(3 * G,),
        in_specs=[
            pl.BlockSpec((None, B, C, HW), x_map),
            pl.BlockSpec((2 * C + 8, 9 * C), lambda i: (0, 0)),
        ],
        out_specs=pl.BlockSpec((None, B, C, HW), out_map),
        out_shape=jax.ShapeDtypeStruct((G, B, C, HW), jnp.float32),
        scratch_shapes=[
            pltpu.VMEM((G, B, C, HW), _BF),
            pltpu.VMEM((G, B, C, HW), _BF),
            pltpu.VMEM((G, B, C, HW), _BF),
            pltpu.VMEM((C, 2), jnp.float32),
            pltpu.VMEM((C, 2), jnp.float32),
            pltpu.VMEM((C, 4), _BF),
            pltpu.VMEM((C, 2), jnp.float32),
        ],
        compiler_params=pltpu.CompilerParams(
            dimension_semantics=("arbitrary",),
            vmem_limit_bytes=(58 << 20) + (1 << 19),
        ),
    )

    out = mono(xg, w_all)
    return out.reshape(N, C, H, W)
